# baseline (speedup 1.0000x reference)
import functools

import jax
import jax.numpy as jnp
from jax import lax
from jax.experimental import pallas as pl
from jax.experimental.pallas import tpu as pltpu
from jax.experimental.pallas import tpu_sc as plsc

BATCH = 4096
NUM_FIELDS = 26
EMBED_DIM = 16
FIELD_SIZE = 40000
NUM_WORKERS = 32
B_PER_W = BATCH // NUM_WORKERS
LOOKUPS_PER_W = B_PER_W * NUM_FIELDS
CHUNK = 128
NUM_CHUNKS = LOOKUPS_PER_W // CHUNK
OUT_PER_W = B_PER_W * EMBED_DIM * NUM_FIELDS
LANES = 16


def _emb_body(x_hbm, table_hbm, out_hbm, x_v, idx_v, rows_v, out_v, sem):
    nc = 2
    wid = lax.axis_index("s") * nc + lax.axis_index("c")

    pltpu.sync_copy(x_hbm.at[pl.ds(wid * LOOKUPS_PER_W, LOOKUPS_PER_W)], x_v)

    iota = lax.broadcasted_iota(jnp.int32, (LANES,), 0)

    copies = []
    for c in range(NUM_CHUNKS):
        for q in range(CHUNK // LANES):
            base = c * CHUNK + q * LANES
            p = base + iota
            f = lax.rem(p, NUM_FIELDS)
            idx_v[c, pl.ds(q * LANES, LANES)] = (
                x_v[pl.ds(base, LANES)] + f * FIELD_SIZE
            )
        copies.append(
            pltpu.make_async_copy(
                table_hbm.at[idx_v.at[c]],
                rows_v.at[pl.ds(c * CHUNK, CHUNK)],
                sem,
            )
        )
        copies[-1].start()
    for cp in copies:
        cp.wait()

    c26 = iota * NUM_FIELDS

    def transpose_b(b, carry):
        bb = b * (EMBED_DIM * NUM_FIELDS) + c26
        for f in range(NUM_FIELDS):
            vals = rows_v[b * NUM_FIELDS + f]
            plsc.store_scatter(out_v, [bb + f], vals)
        return carry

    lax.fori_loop(0, B_PER_W, transpose_b, None, unroll=2)

    pltpu.sync_copy(out_v, out_hbm.at[pl.ds(wid * OUT_PER_W, OUT_PER_W)])


@jax.jit
def kernel(x, table):
    x_flat = x.reshape(-1)
    mesh = plsc.VectorSubcoreMesh(core_axis_name="c", subcore_axis_name="s")
    out_flat = pl.kernel(
        _emb_body,
        out_type=jax.ShapeDtypeStruct((BATCH * EMBED_DIM * NUM_FIELDS,), jnp.float32),
        mesh=mesh,
        compiler_params=pltpu.CompilerParams(
            needs_layout_passes=False, use_tc_tiling_on_sc=False
        ),
        scratch_types=[
            pltpu.VMEM((LOOKUPS_PER_W,), jnp.int32),
            pltpu.VMEM((NUM_CHUNKS, CHUNK), jnp.int32),
            pltpu.VMEM((LOOKUPS_PER_W, EMBED_DIM), jnp.float32),
            pltpu.VMEM((OUT_PER_W,), jnp.float32),
            pltpu.SemaphoreType.DMA,
        ],
    )(x_flat, table)
    return out_flat.reshape(BATCH, EMBED_DIM, NUM_FIELDS)


# speedup vs baseline: 6.4875x; 6.4875x over previous
import jax
import jax.numpy as jnp
from jax import lax
from jax.experimental import pallas as pl
from jax.experimental.pallas import tpu as pltpu
from jax.experimental.pallas import tpu_sc as plsc

BATCH = 4096
NUM_FIELDS = 26
EMBED_DIM = 16
FIELD_SIZE = 40000
NUM_ROWS = FIELD_SIZE * NUM_FIELDS
NUM_WORKERS = 32
B_PER_W = BATCH // NUM_WORKERS
X_PER_W = B_PER_W * NUM_FIELDS
LANES = 16
HALF = NUM_ROWS * 8
ELEMS_PER_F = 16 * 128
OUT_PER_W = NUM_FIELDS * ELEMS_PER_F
F_STRIDE = EMBED_DIM * BATCH
RB_STRIDE = 8 * BATCH


def _emb_body(x_hbm, tab_hbm, out_hbm, x_v, idx_v, out_v, sem0, sem1, sem2):
    nc = 2
    wid = lax.axis_index("s") * nc + lax.axis_index("c")
    sems = (sem0, sem1)

    pltpu.sync_copy(x_hbm.at[pl.ds(wid * X_PER_W, X_PER_W)], x_v)

    iota = lax.broadcasted_iota(jnp.int32, (LANES,), 0)

    def compute_and_fire(f, sem):
        foff = f * FIELD_SIZE
        for q in range(B_PER_W // LANES):
            lanes = q * LANES + iota
            xv = plsc.load_gather(x_v, [lanes * NUM_FIELDS + f])
            idx = xv + foff
            base = (
                lax.shift_right_logical(idx, 7) * 1024
                + lax.bitwise_and(idx, 127)
            )
            for rb in range(2):
                for d8 in range(8):
                    idx_v[f, rb * 8 + d8, pl.ds(q * LANES, LANES)] = (
                        base + (rb * HALF + d8 * 128)
                    )
        for r in range(16):
            pltpu.make_async_copy(
                tab_hbm.at[idx_v.at[f, r]],
                out_v.at[pl.ds(f * ELEMS_PER_F + r * 128, 128)],
                sem,
            ).start()

    def drain_and_store(f, sem):
        pltpu.make_async_copy(
            tab_hbm.at[pl.ds(0, ELEMS_PER_F)],
            out_v.at[pl.ds(f * ELEMS_PER_F, ELEMS_PER_F)],
            sem,
        ).wait()
        for rb in range(2):
            pltpu.make_async_copy(
                out_v.at[pl.ds(f * ELEMS_PER_F + rb * 1024, 1024)],
                out_hbm.at[
                    pl.ds(f * F_STRIDE + rb * RB_STRIDE + wid * 1024, 1024)
                ],
                sem2,
            ).start()

    def group(g, carry):
        for j in range(2):
            f = g * 2 + j

            @pl.when(g > 0)
            def _():
                drain_and_store(f - 2, sems[j])

            compute_and_fire(f, sems[j])
        return carry

    lax.fori_loop(0, NUM_FIELDS // 2, group, None)
    for j in range(2):
        drain_and_store(NUM_FIELDS - 2 + j, sems[j])

    pltpu.make_async_copy(out_v, out_hbm.at[pl.ds(0, OUT_PER_W)], sem2).wait()


@jax.jit
def kernel(x, table):
    x_flat = x.reshape(-1)
    tab_flat = (
        table.T.reshape(2, 8, NUM_ROWS // 128, 128)
        .transpose(0, 2, 1, 3)
        .reshape(-1)
    )
    mesh = plsc.VectorSubcoreMesh(core_axis_name="c", subcore_axis_name="s")
    out_flat = pl.kernel(
        _emb_body,
        out_type=jax.ShapeDtypeStruct((BATCH * EMBED_DIM * NUM_FIELDS,), jnp.float32),
        mesh=mesh,
        compiler_params=pltpu.CompilerParams(
            needs_layout_passes=False, use_tc_tiling_on_sc=False
        ),
        scratch_types=[
            pltpu.VMEM((X_PER_W,), jnp.int32),
            pltpu.VMEM((NUM_FIELDS, 16, 128), jnp.int32),
            pltpu.VMEM((OUT_PER_W,), jnp.float32),
            pltpu.SemaphoreType.DMA,
            pltpu.SemaphoreType.DMA,
            pltpu.SemaphoreType.DMA,
        ],
    )(x_flat, tab_flat)
    return (
        out_flat.reshape(NUM_FIELDS, 2, NUM_WORKERS, 8, 128)
        .transpose(0, 1, 3, 2, 4)
        .reshape(NUM_FIELDS, EMBED_DIM, BATCH)
        .transpose(2, 1, 0)
    )


# speedup vs baseline: 6.7299x; 1.0374x over previous
import jax
import jax.numpy as jnp
from jax import lax
from jax.experimental import pallas as pl
from jax.experimental.pallas import tpu as pltpu
from jax.experimental.pallas import tpu_sc as plsc

BATCH = 4096
NUM_FIELDS = 26
EMBED_DIM = 16
FIELD_SIZE = 40000
NUM_ROWS = FIELD_SIZE * NUM_FIELDS
NUM_WORKERS = 32
B_PER_W = BATCH // NUM_WORKERS
X_PER_W = B_PER_W * NUM_FIELDS
LANES = 16
HALF = NUM_ROWS * 8
ROWS_PER_W = NUM_FIELDS * 2 * 8
OUT_PER_W = ROWS_PER_W * 128
F_STRIDE = EMBED_DIM * BATCH
RB_STRIDE = 8 * BATCH


def _emb_body(x_hbm, tab_hbm, out_hbm, x_v, idx_v, out_v, sem, sem2):
    nc = 2
    wid = lax.axis_index("s") * nc + lax.axis_index("c")

    pltpu.sync_copy(x_hbm.at[pl.ds(wid * X_PER_W, X_PER_W)], x_v)

    iota = lax.broadcasted_iota(jnp.int32, (LANES,), 0)

    def per_field(f, carry):
        foff = f * FIELD_SIZE
        for q in range(B_PER_W // LANES):
            lanes = q * LANES + iota
            xv = plsc.load_gather(x_v, [lanes * NUM_FIELDS + f])
            idx = xv + foff
            base = (
                lax.shift_right_logical(idx, 7) * 1024
                + lax.bitwise_and(idx, 127)
            )
            for rb in range(2):
                for d8 in range(8):
                    row = f * 16 + rb * 8 + d8
                    idx_v[row, pl.ds(q * LANES, LANES)] = (
                        base + (rb * HALF + d8 * 128)
                    )
        for r in range(16):
            row = f * 16 + r
            pltpu.make_async_copy(
                tab_hbm.at[idx_v.at[row]],
                out_v.at[pl.ds(row * 128, 128)],
                sem,
            ).start()
        return carry

    lax.fori_loop(0, NUM_FIELDS, per_field, None, unroll=2)

    pltpu.make_async_copy(
        tab_hbm.at[pl.ds(0, OUT_PER_W)], out_v, sem
    ).wait()

    def store_field(f, carry):
        for rb in range(2):
            pltpu.make_async_copy(
                out_v.at[pl.ds((f * 2 + rb) * 1024, 1024)],
                out_hbm.at[
                    pl.ds(f * F_STRIDE + rb * RB_STRIDE + wid * 1024, 1024)
                ],
                sem2,
            ).start()
        return carry

    lax.fori_loop(0, NUM_FIELDS, store_field, None)
    pltpu.make_async_copy(out_v, out_hbm.at[pl.ds(0, OUT_PER_W)], sem2).wait()


@jax.jit
def kernel(x, table):
    x_flat = x.reshape(-1)
    tab_flat = (
        table.T.reshape(2, 8, NUM_ROWS // 128, 128)
        .transpose(0, 2, 1, 3)
        .reshape(-1)
    )
    mesh = plsc.VectorSubcoreMesh(core_axis_name="c", subcore_axis_name="s")
    out_flat = pl.kernel(
        _emb_body,
        out_type=jax.ShapeDtypeStruct((BATCH * EMBED_DIM * NUM_FIELDS,), jnp.float32),
        mesh=mesh,
        compiler_params=pltpu.CompilerParams(
            needs_layout_passes=False, use_tc_tiling_on_sc=False
        ),
        scratch_types=[
            pltpu.VMEM((X_PER_W,), jnp.int32),
            pltpu.VMEM((ROWS_PER_W, 128), jnp.int32),
            pltpu.VMEM((OUT_PER_W,), jnp.float32),
            pltpu.SemaphoreType.DMA,
            pltpu.SemaphoreType.DMA,
        ],
    )(x_flat, tab_flat)
    return (
        out_flat.reshape(NUM_FIELDS, 2, NUM_WORKERS, 8, 128)
        .transpose(0, 1, 3, 2, 4)
        .reshape(NUM_FIELDS, EMBED_DIM, BATCH)
        .transpose(2, 1, 0)
    )
